# Initial kernel scaffold; baseline (speedup 1.0000x reference)
import jax
import jax.numpy as jnp
from jax.experimental import pallas as pl


def kernel(input, W):
    raise NotImplementedError("write your pallas kernel here")


# speedup vs baseline: 26.1578x; 26.1578x over previous
import jax
import jax.numpy as jnp
from jax import lax
from jax.experimental import pallas as pl
from jax.experimental.pallas import tpu as pltpu
from jax.experimental.pallas import tpu_sc as plsc

NGRAMS = 20
BATCH = 1024
EMB_DIM = 64
LANES = 16
NW = 32
BPW = BATCH // NW
IDX_PER_W = BPW * NGRAMS
GCHUNK = 128
NCHUNK = IDX_PER_W // GCHUNK
ZROW = IDX_PER_W
NGROUP = BPW // LANES


def _sc_body(idx_hbm, table_hbm, out_hbm, idx_v, rows_v, out_v, sem):
    wid = lax.axis_index("s") * 2 + lax.axis_index("c")
    base = wid * BPW

    pltpu.sync_copy(idx_hbm.at[pl.ds(base * NGRAMS, IDX_PER_W)], idx_v)

    copies = [
        pltpu.make_async_copy(
            table_hbm.at[idx_v.at[pl.ds(j * GCHUNK, GCHUNK)]],
            rows_v.at[pl.ds(j * GCHUNK, GCHUNK)],
            sem,
        )
        for j in range(NCHUNK)
    ]
    for c in copies:
        c.start()

    zeros16 = jnp.zeros((LANES,), jnp.float32)
    for dc in range(EMB_DIM // LANES):
        rows_v[ZROW, pl.ds(dc * LANES, LANES)] = zeros16

    lanes = lax.iota(jnp.int32, LANES)

    groups = []
    for g in range(NGROUP):
        p = [lanes * NGRAMS + (g * LANES * NGRAMS + s) for s in range(NGRAMS)]
        v = [plsc.load_gather(idx_v, [p[s]]) for s in range(NGRAMS)]
        pf = [p[0]]
        for s in range(1, NGRAMS):
            dup = v[s] == v[0]
            for t in range(1, s):
                dup = dup | (v[s] == v[t])
            pf.append(jnp.where(dup, ZROW, p[s]))
        groups.append(pf)

    for c in copies:
        c.wait()

    for g in range(NGROUP):
        pf = groups[g]
        row_out = lanes + g * LANES

        def dbody(d, _):
            col = jnp.full((LANES,), d, jnp.int32)
            acc = plsc.load_gather(rows_v, [pf[0], col])
            for s in range(1, NGRAMS):
                acc = acc + plsc.load_gather(rows_v, [pf[s], col])
            plsc.store_scatter(out_v, [row_out, col], acc)
            return _

        lax.fori_loop(0, EMB_DIM, dbody, None, unroll=4)

    pltpu.sync_copy(out_v, out_hbm.at[pl.ds(base, BPW)])


def kernel(input, W):
    idx_flat = input.T.reshape(-1)
    mesh = plsc.VectorSubcoreMesh(core_axis_name="c", subcore_axis_name="s")
    f = pl.kernel(
        _sc_body,
        out_type=jax.ShapeDtypeStruct((BATCH, EMB_DIM), jnp.float32),
        mesh=mesh,
        compiler_params=pltpu.CompilerParams(
            needs_layout_passes=False, use_tc_tiling_on_sc=False
        ),
        scratch_types=[
            pltpu.VMEM((IDX_PER_W,), jnp.int32),
            pltpu.VMEM((IDX_PER_W + 1, EMB_DIM), jnp.float32),
            pltpu.VMEM((BPW, EMB_DIM), jnp.float32),
            pltpu.SemaphoreType.DMA,
        ],
    )
    return f(idx_flat, W)
